# baseline (speedup 1.0000x reference)
import functools

import jax
import jax.numpy as jnp
from jax import lax
from jax.experimental import pallas as pl
from jax.experimental.pallas import tpu as pltpu
from jax.experimental.pallas import tpu_sc as plsc

_B = 4096
_D = 32
_V = 1000000

_info = plsc.get_sparse_core_info()
_NC, _NS = _info.num_cores, _info.num_subcores
_NW = _NC * _NS
_G = 16
_NCHUNK = 4


@functools.cache
def _make_gather(n_idx, transposed):
    bpw = n_idx // _NW
    n_groups = bpw // _G
    out_shape = (_D, n_idx) if transposed else (n_idx, _D)
    buf_shape = (_D, bpw) if transposed else (bpw, _D)
    mesh = plsc.VectorSubcoreMesh(core_axis_name="c", subcore_axis_name="s")

    @functools.partial(
        pl.kernel,
        mesh=mesh,
        compiler_params=pltpu.CompilerParams(
            use_tc_tiling_on_sc=True, needs_layout_passes=False),
        out_type=jax.ShapeDtypeStruct(out_shape, jnp.float32),
        scratch_types=[
            pltpu.VMEM((bpw,), jnp.int32),
            pltpu.VMEM(buf_shape, jnp.float32),
            pltpu.VMEM((_G, _D, 128), jnp.float32),
            pltpu.SemaphoreType.DMA,
            pltpu.SemaphoreType.DMA,
        ],
    )
    def gather_k(idx_hbm, src_hbm, out_hbm, idx_v, col_v, slab, sem_a, sem_b):
        wid = lax.axis_index("s") * _NC + lax.axis_index("c")
        base = pl.multiple_of(wid * bpw, _G)
        pltpu.sync_copy(idx_hbm.at[pl.ds(base, bpw)], idx_v)
        row_lo = lax.iota(jnp.int32, 16)
        row_hi = row_lo + 16

        def issue(vb, slot, sem):
            l128 = pl.multiple_of((vb >> 7) * 128, 128)
            pltpu.async_copy(src_hbm.at[:, pl.ds(l128, 128)], slab.at[slot], sem)

        def extract(vb, j, slot):
            col = jnp.full((16,), vb & 127, jnp.int32)
            jv = jnp.full((16,), j, jnp.int32)
            lo = plsc.load_gather(slab.at[slot], [row_lo, col])
            hi = plsc.load_gather(slab.at[slot], [row_hi, col])
            if transposed:
                plsc.store_scatter(col_v, [row_lo, jv], lo)
                plsc.store_scatter(col_v, [row_hi, jv], hi)
            else:
                plsc.store_scatter(col_v, [jv, row_lo], lo)
                plsc.store_scatter(col_v, [jv, row_hi], hi)

        vv0 = idx_v[pl.ds(0, _G)]
        for b in range(8):
            issue(vv0[b], b, sem_a)
        for b in range(8, 16):
            issue(vv0[b], b, sem_b)

        def group(g, vcur):
            nxt = jnp.minimum((g + 1) * _G, bpw - _G)
            vnxt = idx_v[pl.ds(nxt, _G)]
            not_last = g < n_groups - 1
            for half, sem in ((0, sem_a), (1, sem_b)):
                for b in range(half * 8, half * 8 + 8):
                    pltpu.make_async_copy(
                        src_hbm.at[:, pl.ds(0, 128)], slab.at[b], sem).wait()
                for b in range(half * 8, half * 8 + 8):
                    extract(vcur[b], g * _G + b, b)

                @pl.when(not_last)
                def _():
                    for b in range(half * 8, half * 8 + 8):
                        issue(vnxt[b], b, sem)
            return vnxt

        lax.fori_loop(0, n_groups, group, vv0)
        if transposed:
            pltpu.sync_copy(col_v, out_hbm.at[:, pl.ds(base, bpw)])
        else:
            pltpu.sync_copy(col_v, out_hbm.at[pl.ds(base, bpw), :])

    return gather_k


_BM = 512
_CHUNK = _B // _NCHUNK


def _mm(a_ref, b_ref, o_ref):
    o_ref[...] = lax.dot_general(
        a_ref[...], b_ref[...],
        (((1,), (0,)), ((), ())),
        preferred_element_type=jnp.float32,
    )


def _mm_prev(prev_ref, a_ref, b_ref, o_ref):
    del prev_ref
    _mm(a_ref, b_ref, o_ref)


@functools.cache
def _make_mm(chunk_idx):
    row0 = chunk_idx * (_CHUNK // _BM)
    ab_specs = [
        pl.BlockSpec((_BM, _D), lambda i: (i, 0)),
        pl.BlockSpec((_D, _B), lambda i: (0, 0)),
    ]
    out_spec = pl.BlockSpec((_BM, _B), lambda i: (row0 + i, 0))
    out_shape = jax.ShapeDtypeStruct((_B, _B), jnp.float32)
    if chunk_idx == 0:
        return pl.pallas_call(
            _mm,
            grid=(_CHUNK // _BM,),
            in_specs=ab_specs,
            out_specs=out_spec,
            out_shape=out_shape,
        )
    return pl.pallas_call(
        _mm_prev,
        grid=(_CHUNK // _BM,),
        in_specs=[pl.BlockSpec(memory_space=pl.ANY)] + ab_specs,
        out_specs=out_spec,
        out_shape=out_shape,
        input_output_aliases={0: 0},
    )


def kernel(target, context, in_embed, out_embed):
    target = target.astype(jnp.int32)
    context = context.astype(jnp.int32)
    inT = in_embed.T
    outT = out_embed.T
    bT = _make_gather(_B, True)(context, outT)
    aT = [
        _make_gather(_CHUNK, False)(
            lax.dynamic_slice_in_dim(target, i * _CHUNK, _CHUNK), inT)
        for i in range(_NCHUNK)
    ]
    scores = _make_mm(0)(aT[0], bT)
    for i in range(1, _NCHUNK):
        scores = _make_mm(i)(scores, aT[i], bT)
    return scores


# speedup vs baseline: 1.0566x; 1.0566x over previous
import functools

import jax
import jax.numpy as jnp
from jax import lax
from jax.experimental import pallas as pl
from jax.experimental.pallas import tpu as pltpu
from jax.experimental.pallas import tpu_sc as plsc

_B = 4096
_D = 32
_V = 1000000

_info = plsc.get_sparse_core_info()
_NC, _NS = _info.num_cores, _info.num_subcores
_NW = _NC * _NS
_BPW = _B // _NW
_G = 16


def _make_gather():
    n_groups = _BPW // _G
    mesh = plsc.VectorSubcoreMesh(core_axis_name="c", subcore_axis_name="s")

    @functools.partial(
        pl.kernel,
        mesh=mesh,
        compiler_params=pltpu.CompilerParams(
            use_tc_tiling_on_sc=True, needs_layout_passes=False),
        out_type=(
            jax.ShapeDtypeStruct((_D, _B), jnp.float32),
            jax.ShapeDtypeStruct((_D, _B), jnp.float32),
        ),
        scratch_types=[
            pltpu.VMEM((_BPW,), jnp.int32),
            pltpu.VMEM((_BPW,), jnp.int32),
            pltpu.VMEM((_D, _BPW), jnp.float32),
            pltpu.VMEM((_D, _BPW), jnp.float32),
            pltpu.VMEM((_G, _D, 128), jnp.float32),
            pltpu.SemaphoreType.DMA,
            pltpu.SemaphoreType.DMA,
        ],
    )
    def gather_k(tgt_hbm, ctx_hbm, inT_hbm, outT_hbm, aT_out, bT_out,
                 idx_a, idx_b, at_v, bt_v, slab, sem_a, sem_b):
        wid = lax.axis_index("s") * _NC + lax.axis_index("c")
        base = pl.multiple_of(wid * _BPW, 128)
        pltpu.sync_copy(tgt_hbm.at[pl.ds(base, _BPW)], idx_a)
        pltpu.sync_copy(ctx_hbm.at[pl.ds(base, _BPW)], idx_b)
        row_lo = lax.iota(jnp.int32, 16)
        row_hi = row_lo + 16

        def phase(idx_ref, src_ref, dst_ref):
            def issue(vb, slot, sem):
                l128 = pl.multiple_of((vb >> 7) * 128, 128)
                pltpu.async_copy(
                    src_ref.at[:, pl.ds(l128, 128)], slab.at[slot], sem)

            def extract(vb, j, slot):
                col = jnp.full((16,), vb & 127, jnp.int32)
                jv = jnp.full((16,), j, jnp.int32)
                lo = plsc.load_gather(slab.at[slot], [row_lo, col])
                hi = plsc.load_gather(slab.at[slot], [row_hi, col])
                plsc.store_scatter(dst_ref, [row_lo, jv], lo)
                plsc.store_scatter(dst_ref, [row_hi, jv], hi)

            vv0 = idx_ref[pl.ds(0, _G)]
            for b in range(8):
                issue(vv0[b], b, sem_a)
            for b in range(8, 16):
                issue(vv0[b], b, sem_b)

            def group(g, vcur):
                nxt = jnp.minimum((g + 1) * _G, _BPW - _G)
                vnxt = idx_ref[pl.ds(nxt, _G)]
                not_last = g < n_groups - 1
                for half, sem in ((0, sem_a), (1, sem_b)):
                    for b in range(half * 8, half * 8 + 8):
                        pltpu.make_async_copy(
                            src_ref.at[:, pl.ds(0, 128)], slab.at[b], sem).wait()
                    for b in range(half * 8, half * 8 + 8):
                        extract(vcur[b], g * _G + b, b)

                    @pl.when(not_last)
                    def _():
                        for b in range(half * 8, half * 8 + 8):
                            issue(vnxt[b], b, sem)
                return vnxt

            lax.fori_loop(0, n_groups, group, vv0)

        phase(idx_a, inT_hbm, at_v)
        phase(idx_b, outT_hbm, bt_v)
        pltpu.sync_copy(at_v, aT_out.at[:, pl.ds(base, _BPW)])
        pltpu.sync_copy(bt_v, bT_out.at[:, pl.ds(base, _BPW)])

    return gather_k


_gather = _make_gather()

_BM = 512
_NSTRIPE = 4
_SPS = _B // _NSTRIPE // _BM


def _mm(a_ref, b_ref, o_ref):
    o_ref[...] = lax.dot_general(
        a_ref[...], b_ref[...],
        (((0,), (0,)), ((), ())),
        preferred_element_type=jnp.float32,
    )


def _mm_prev(prev_ref, a_ref, b_ref, o_ref):
    del prev_ref
    _mm(a_ref, b_ref, o_ref)


@functools.cache
def _make_mm(stripe):
    row0 = stripe * _SPS
    ab_specs = [
        pl.BlockSpec((_D, _BM), lambda i: (0, row0 + i)),
        pl.BlockSpec((_D, _B), lambda i: (0, 0)),
    ]
    out_spec = pl.BlockSpec((_BM, _B), lambda i: (row0 + i, 0))
    out_shape = jax.ShapeDtypeStruct((_B, _B), jnp.float32)
    if stripe == 0:
        return pl.pallas_call(
            _mm,
            grid=(_SPS,),
            in_specs=ab_specs,
            out_specs=out_spec,
            out_shape=out_shape,
        )
    return pl.pallas_call(
        _mm_prev,
        grid=(_SPS,),
        in_specs=[pl.BlockSpec(memory_space=pl.ANY)] + ab_specs,
        out_specs=out_spec,
        out_shape=out_shape,
        input_output_aliases={0: 0},
    )


def kernel(target, context, in_embed, out_embed):
    aT, bT = _gather(
        target.astype(jnp.int32), context.astype(jnp.int32),
        in_embed.T, out_embed.T,
    )
    scores = _make_mm(0)(aT, bT)
    for i in range(1, _NSTRIPE):
        scores = _make_mm(i)(scores, aT, bT)
    return scores
